# Initial kernel scaffold; baseline (speedup 1.0000x reference)
import jax
import jax.numpy as jnp
from jax.experimental import pallas as pl


def kernel(adj, edge_index, edge_val, embedding, channel, W_item0, W_item1, W_i1, W_i2):
    raise NotImplementedError("write your pallas kernel here")


# speedup vs baseline: 3.5713x; 3.5713x over previous
import functools

import jax
import jax.numpy as jnp
from jax import lax
from jax.experimental import pallas as pl
from jax.experimental.pallas import tpu as pltpu
from jax.experimental.pallas import tpu_sc as plsc

_PCALL = pl.pallas_call

_N = 50000
_E = 800000
_EMB = 100
_P = 128
_CC = 4
_CW = 32
_NC = 2
_NS = 16
_NP = 51200
_EPAD = 802816
_R128 = _EPAD // 128
_TILE_E = _EPAD // _NS
_SUP = 512
_NSUB = _SUP // 128
_NSUP = _TILE_E // _SUP
_ROWS_T = _NP // _NS
_ZB = 160
_NZ = _ROWS_T // _ZB
_ROWB = 2000
_NBLK = _N // _ROWB


def _sc_edge_segsum(table, col1d, row2d, val1d):
    mesh = plsc.VectorSubcoreMesh(
        core_axis_name="c", subcore_axis_name="s",
        num_cores=_NC, num_subcores=_NS)

    @functools.partial(
        pl.kernel,
        out_type=jax.ShapeDtypeStruct((_CC * _NP, _CW), jnp.float32),
        mesh=mesh,
        compiler_params=pltpu.CompilerParams(use_tc_tiling_on_sc=False),
        scratch_types=[
            pltpu.VMEM((_SUP,), jnp.int32),
            pltpu.VMEM((8, 128), jnp.int32),
            pltpu.VMEM((_SUP,), jnp.float32),
            pltpu.VMEM((_SUP, _CW), jnp.float32),
            pltpu.VMEM((_ZB, _CW), jnp.float32),
            pltpu.VMEM_SHARED((_NP, _CW), jnp.float32),
            pltpu.SemaphoreType.DMA,
        ],
    )
    def k(table_h, col_h, row_h, val_h, out_h, colv, rowv, valv, rows, zb,
          acc, sem):
        core = lax.axis_index("c")
        sub = lax.axis_index("s")
        zvec = jnp.zeros((16,), jnp.float32)

        def zfill(r, _):
            zb[r, pl.ds(0, 16)] = zvec
            zb[r, pl.ds(16, 16)] = zvec
            return 0
        lax.fori_loop(0, _ZB, zfill, 0)

        for j in range(_CC // _NC):
            cc = core * (_CC // _NC) + j
            for z in range(_NZ):
                pltpu.sync_copy(zb, acc.at[pl.ds(sub * _ROWS_T + z * _ZB, _ZB)])
            plsc.subcore_barrier()

            ccvec = jnp.full((16,), cc * _NP, jnp.int32)

            def body(sb, _):
                ebase = sub * _TILE_E + sb * _SUP
                pltpu.sync_copy(col_h.at[pl.ds(ebase, _SUP)], colv)
                pltpu.sync_copy(val_h.at[pl.ds(ebase, _SUP)], valv)

                @pl.when(sb % 2 == 0)
                def _load_rows():
                    rbase = sub * (_TILE_E // 128) + (sb // 2) * 8
                    pltpu.sync_copy(row_h.at[pl.ds(rbase, 8)], rowv)

                for k16 in range(_SUP // 16):
                    colv[pl.ds(k16 * 16, 16)] = colv[pl.ds(k16 * 16, 16)] + ccvec

                descs = [
                    pltpu.async_copy(table_h.at[colv.at[pl.ds(a * 128, 128)]],
                                     rows.at[pl.ds(a * 128, 128)], sem)
                    for a in range(_NSUB)]
                for d in descs:
                    d.wait()

                def scale(g, _):
                    v16 = valv[pl.ds(g * 16, 16)]
                    for l in range(16):
                        e = g * 16 + l
                        b = jnp.full((16,), v16[l], jnp.float32)
                        rows[e, pl.ds(0, 16)] = rows[e, pl.ds(0, 16)] * b
                        rows[e, pl.ds(16, 16)] = rows[e, pl.ds(16, 16)] * b
                    return 0
                lax.fori_loop(0, _SUP // 16, scale, 0)

                half = (sb % 2) * _NSUB
                for a in range(_NSUB):
                    pltpu.sync_copy(rows.at[pl.ds(a * 128, 128)],
                                    acc.at[rowv.at[half + a]], add=True)
                return 0
            lax.fori_loop(0, _NSUP, body, 0)
            plsc.subcore_barrier()

            obase = cc * _NP + sub * _ROWS_T
            for z in range(_NZ):
                pltpu.sync_copy(acc.at[pl.ds(sub * _ROWS_T + z * _ZB, _ZB)],
                                out_h.at[pl.ds(obase + z * _ZB, _ZB)])

    return k(table, col1d, row2d, val1d)


def _cat4(xs_ref):
    return jnp.concatenate([xs_ref[c] for c in range(_CC)], axis=1)


def _tc_matmul_chunks(x, w):
    def body(x_ref, w_ref, out_ref):
        xw = jnp.dot(x_ref[...], w_ref[...],
                     preferred_element_type=jnp.float32)
        for c in range(_CC):
            out_ref[c] = xw[:, c * _CW:(c + 1) * _CW]
    return _PCALL(
        body,
        grid=(_NBLK,),
        in_specs=[pl.BlockSpec((_ROWB, _P), lambda r: (r, 0)),
                  pl.BlockSpec((_P, _P), lambda r: (0, 0))],
        out_specs=pl.BlockSpec((_CC, _ROWB, _CW), lambda r: (0, r, 0)),
        out_shape=jax.ShapeDtypeStruct((_CC, _NP, _CW), jnp.float32),
    )(x, w)


def _tc_c1(xs4, adjT, wi1, wi2):
    def body(xs_ref, adjt_ref, wi1_ref, wi2_ref, H1_ref, h1_ref):
        xsb = _cat4(xs_ref)
        t = jnp.dot(xsb, wi1_ref[...], preferred_element_type=jnp.float32)
        t = jnp.maximum(t + xsb, 0.0)
        logits = jnp.dot(t, wi2_ref[...], preferred_element_type=jnp.float32)
        cols = lax.broadcasted_iota(jnp.int32, logits.shape, 1)
        logits = jnp.where(cols < _EMB, logits, -1e30)
        m = jnp.max(logits, axis=1, keepdims=True)
        p = jnp.exp(logits - m)
        H1 = p / jnp.sum(p, axis=1, keepdims=True)
        H1_ref[...] = H1
        g = H1 * adjt_ref[...]
        s = jnp.sum(g, axis=1, keepdims=True)
        gn = g / (s + 1e-8)
        part = lax.dot_general(gn, xsb, (((0,), (0,)), ((), ())),
                               preferred_element_type=jnp.float32)

        @pl.when(pl.program_id(0) == 0)
        def _init():
            h1_ref[...] = jnp.zeros_like(h1_ref)
        h1_ref[...] += part

    return _PCALL(
        body,
        grid=(_NBLK,),
        in_specs=[pl.BlockSpec((_CC, _ROWB, _CW), lambda r: (0, r, 0)),
                  pl.BlockSpec((_ROWB, _P), lambda r: (r, 0)),
                  pl.BlockSpec((_P, _P), lambda r: (0, 0)),
                  pl.BlockSpec((_P, _P), lambda r: (0, 0))],
        out_specs=[pl.BlockSpec((_ROWB, _P), lambda r: (r, 0)),
                   pl.BlockSpec((_P, _P), lambda r: (0, 0))],
        out_shape=[jax.ShapeDtypeStruct((_N, _P), jnp.float32),
                   jax.ShapeDtypeStruct((_P, _P), jnp.float32)],
    )(xs4, adjT, wi1, wi2)


def _nrm(v):
    n = jnp.sqrt(jnp.sum(v * v, axis=1, keepdims=True))
    return v / jnp.maximum(n, 1e-12)


def _tc_c2_fused(H1, h1, xs4, acci, acch, wnext):
    def body(H1_ref, h1_ref, xs_ref, acci_ref, acch_ref, wn_ref,
             xw_ref, accio_ref, accho_ref):
        xsb = _cat4(xs_ref)
        h2 = jnp.dot(H1_ref[...], h1_ref[...],
                     preferred_element_type=jnp.float32)
        xo = h2 + xsb
        accio_ref[...] = acci_ref[...] + _nrm(xo)
        accho_ref[...] = acch_ref[...] + _nrm(h2)
        xw = jnp.dot(xo, wn_ref[...], preferred_element_type=jnp.float32)
        for c in range(_CC):
            xw_ref[c] = xw[:, c * _CW:(c + 1) * _CW]

    return _PCALL(
        body,
        grid=(_NBLK,),
        in_specs=[pl.BlockSpec((_ROWB, _P), lambda r: (r, 0)),
                  pl.BlockSpec((_P, _P), lambda r: (0, 0)),
                  pl.BlockSpec((_CC, _ROWB, _CW), lambda r: (0, r, 0)),
                  pl.BlockSpec((_ROWB, _P), lambda r: (r, 0)),
                  pl.BlockSpec((_ROWB, _P), lambda r: (r, 0)),
                  pl.BlockSpec((_P, _P), lambda r: (0, 0))],
        out_specs=[pl.BlockSpec((_CC, _ROWB, _CW), lambda r: (0, r, 0)),
                   pl.BlockSpec((_ROWB, _P), lambda r: (r, 0)),
                   pl.BlockSpec((_ROWB, _P), lambda r: (r, 0))],
        out_shape=[jax.ShapeDtypeStruct((_CC, _NP, _CW), jnp.float32),
                   jax.ShapeDtypeStruct((_N, _P), jnp.float32),
                   jax.ShapeDtypeStruct((_N, _P), jnp.float32)],
    )(H1, h1, xs4, acci, acch, wnext)


def _tc_c2_final(H1, h1, xs4, acci, acch):
    def body(H1_ref, h1_ref, xs_ref, acci_ref, acch_ref,
             item_ref, hs_ref):
        xsb = _cat4(xs_ref)
        h2 = jnp.dot(H1_ref[...], h1_ref[...],
                     preferred_element_type=jnp.float32)
        xo = h2 + xsb
        item_ref[...] = (acci_ref[...] + _nrm(xo)) * (1.0 / 3.0)
        hs_ref[...] = (acch_ref[...] + _nrm(h2)) * 0.5

    return _PCALL(
        body,
        grid=(_NBLK,),
        in_specs=[pl.BlockSpec((_ROWB, _P), lambda r: (r, 0)),
                  pl.BlockSpec((_P, _P), lambda r: (0, 0)),
                  pl.BlockSpec((_CC, _ROWB, _CW), lambda r: (0, r, 0)),
                  pl.BlockSpec((_ROWB, _P), lambda r: (r, 0)),
                  pl.BlockSpec((_ROWB, _P), lambda r: (r, 0))],
        out_specs=[pl.BlockSpec((_ROWB, _P), lambda r: (r, 0)),
                   pl.BlockSpec((_ROWB, _P), lambda r: (r, 0))],
        out_shape=[jax.ShapeDtypeStruct((_N, _P), jnp.float32),
                   jax.ShapeDtypeStruct((_N, _P), jnp.float32)],
    )(H1, h1, xs4, acci, acch)


def _pad2(w):
    return jnp.pad(w.astype(jnp.float32),
                   ((0, _P - w.shape[0]), (0, _P - w.shape[1])))


def kernel(adj, edge_index, edge_val, embedding, channel,
           W_item0, W_item1, W_i1, W_i2):
    del channel
    f32 = jnp.float32
    emb = jnp.pad(embedding.astype(f32), ((0, 0), (0, _P - _EMB)))
    W0p, W1p = _pad2(W_item0), _pad2(W_item1)
    Wi1p, Wi2p = _pad2(W_i1), _pad2(W_i2)
    adjT = jnp.pad(adj.T.astype(f32), ((0, 0), (0, _P - adj.shape[0])))

    rowp = jnp.pad(edge_index[0], (0, _EPAD - _E)).reshape(_R128, 128)
    colp = jnp.pad(edge_index[1], (0, _EPAD - _E))
    valp = jnp.pad(edge_val.astype(f32), (0, _EPAD - _E))

    acci = emb
    acch = jnp.zeros_like(emb)
    xw = _tc_matmul_chunks(emb, W0p)
    item = hs = None
    for i in range(2):
        xs4 = _sc_edge_segsum(xw.reshape(_CC * _NP, _CW), colp, rowp, valp)
        xs4 = xs4.reshape(_CC, _NP, _CW)
        H1, h1 = _tc_c1(xs4, adjT, Wi1p, Wi2p)
        if i == 0:
            xw, acci, acch = _tc_c2_fused(H1, h1, xs4, acci, acch, W1p)
        else:
            item, hs = _tc_c2_final(H1, h1, xs4, acci, acch)
    return item[:, :_EMB], hs[:, :_EMB]


# speedup vs baseline: 3.8490x; 1.0778x over previous
import functools

import jax
import jax.numpy as jnp
from jax import lax
from jax.experimental import pallas as pl
from jax.experimental.pallas import tpu as pltpu
from jax.experimental.pallas import tpu_sc as plsc

_PCALL = pl.pallas_call

_N = 50000
_E = 800000
_EMB = 100
_P = 128
_CC = 4
_CW = 32
_NC = 2
_NS = 16
_NP = 51200
_EPAD = 802816
_R128 = _EPAD // 128
_TILE_E = _EPAD // _NS
_SUP = 256
_GRP = 1024
_NSUP = _TILE_E // _SUP
_NGRP = _TILE_E // _GRP
_TILE_R8 = _TILE_E // 128
_ROWS_T = _NP // _NS
_ZB = 160
_NZ = _ROWS_T // _ZB
_ROWB = 2000
_NBLK = _N // _ROWB


def _sc_edge_segsum(table, col1d, row2d, val1d):
    mesh = plsc.VectorSubcoreMesh(
        core_axis_name="c", subcore_axis_name="s",
        num_cores=_NC, num_subcores=_NS)

    @functools.partial(
        pl.kernel,
        out_type=jax.ShapeDtypeStruct((_CC * _NP, _CW), jnp.float32),
        mesh=mesh,
        compiler_params=pltpu.CompilerParams(use_tc_tiling_on_sc=False),
        scratch_types=[
            [pltpu.VMEM((_GRP,), jnp.int32)] * 2,
            [pltpu.VMEM((8, 128), jnp.int32)] * 2,
            [pltpu.VMEM((_GRP,), jnp.float32)] * 2,
            [pltpu.VMEM((_SUP, _CW), jnp.float32)] * 2,
            pltpu.VMEM((_ZB, _CW), jnp.float32),
            pltpu.VMEM_SHARED((_NP, _CW), jnp.float32),
            [pltpu.SemaphoreType.DMA] * 2,
            pltpu.SemaphoreType.DMA,
        ],
    )
    def k(table_h, col_h, row_h, val_h, out_h, colv, rowv, valv, rows, zb,
          acc, semg, semi):
        core = lax.axis_index("c")
        sub = lax.axis_index("s")
        zvec = jnp.zeros((16,), jnp.float32)

        def zfill(r, _):
            zb[r, pl.ds(0, 16)] = zvec
            zb[r, pl.ds(16, 16)] = zvec
            return 0
        lax.fori_loop(0, _ZB, zfill, 0)

        def add_cc(colv_b, ccvec):
            for k16 in range(_GRP // 16):
                colv_b[pl.ds(k16 * 16, 16)] = (
                    colv_b[pl.ds(k16 * 16, 16)] + ccvec)

        def fire_gather(p, q, off):
            for a in range(2):
                pltpu.async_copy(
                    table_h.at[colv[q].at[pl.ds(off + a * 128, 128)]],
                    rows[p].at[pl.ds(a * 128, 128)], semg[p])

        def consume(p, q, w):
            pltpu.make_async_copy(table_h.at[pl.ds(0, _SUP)], rows[p],
                                  semg[p]).wait()

            def scale(k16, _):
                v16 = valv[q][pl.ds(w * _SUP + k16 * 16, 16)]
                for l in range(16):
                    e = k16 * 16 + l
                    b = jnp.full((16,), v16[l], jnp.float32)
                    rows[p][e, pl.ds(0, 16)] = rows[p][e, pl.ds(0, 16)] * b
                    rows[p][e, pl.ds(16, 16)] = rows[p][e, pl.ds(16, 16)] * b
                return 0
            lax.fori_loop(0, _SUP // 16, scale, 0)

            for a in range(2):
                pltpu.sync_copy(rows[p].at[pl.ds(a * 128, 128)],
                                acc.at[rowv[q].at[w * 2 + a]], add=True)

        for j in range(_CC // _NC):
            cc = core * (_CC // _NC) + j
            for z in range(_NZ):
                pltpu.sync_copy(zb, acc.at[pl.ds(sub * _ROWS_T + z * _ZB, _ZB)])
            plsc.subcore_barrier()

            ccvec = jnp.full((16,), cc * _NP, jnp.int32)
            ebase0 = sub * _TILE_E
            rbase0 = sub * _TILE_R8

            pltpu.sync_copy(col_h.at[pl.ds(ebase0, _GRP)], colv[0])
            pltpu.sync_copy(val_h.at[pl.ds(ebase0, _GRP)], valv[0])
            pltpu.sync_copy(row_h.at[pl.ds(rbase0, 8)], rowv[0])
            add_cc(colv[0], ccvec)
            fire_gather(0, 0, 0)

            def body(sb, _):
                w = sb % 4
                g = sb // 4
                for p in range(2):
                    for q in range(2):
                        pl.when(((sb % 2) == p) & ((g % 2) == q))(
                            functools.partial(consume, p, q, w))

                for q in range(2):
                    cond = (w == 0) & (sb < (_NSUP - 4)) & (((g + 1) % 2) == q)

                    def _fire_idx(q=q):
                        eb = ebase0 + (g + 1) * _GRP
                        pltpu.async_copy(col_h.at[pl.ds(eb, _GRP)],
                                         colv[q], semi)
                        pltpu.async_copy(val_h.at[pl.ds(eb, _GRP)],
                                         valv[q], semi)
                        rb = rbase0 + (g + 1) * 8
                        pltpu.async_copy(row_h.at[pl.ds(rb, 8)], rowv[q], semi)
                    pl.when(cond)(_fire_idx)

                for q in range(2):
                    cond = ((w == 3) & (sb < (_NSUP - 1))
                            & (((g + 1) % 2) == q))

                    def _drain_idx(q=q):
                        pltpu.make_async_copy(col_h.at[pl.ds(0, _GRP)],
                                              colv[q], semi).wait()
                        pltpu.make_async_copy(val_h.at[pl.ds(0, _GRP)],
                                              valv[q], semi).wait()
                        pltpu.make_async_copy(row_h.at[pl.ds(0, 8)],
                                              rowv[q], semi).wait()
                        add_cc(colv[q], ccvec)
                    pl.when(cond)(_drain_idx)

                for p in range(2):
                    for q in range(2):
                        cond = ((sb < (_NSUP - 1))
                                & (((sb + 1) % 2) == p)
                                & ((((sb + 1) // 4) % 2) == q))
                        off = ((sb + 1) % 4) * _SUP
                        pl.when(cond)(functools.partial(fire_gather, p, q, off))
                return 0
            lax.fori_loop(0, _NSUP, body, 0)
            plsc.subcore_barrier()

            obase = cc * _NP + sub * _ROWS_T
            for z in range(_NZ):
                pltpu.sync_copy(acc.at[pl.ds(sub * _ROWS_T + z * _ZB, _ZB)],
                                out_h.at[pl.ds(obase + z * _ZB, _ZB)])

    return k(table, col1d, row2d, val1d)


def _cat4(xs_ref):
    return jnp.concatenate([xs_ref[c] for c in range(_CC)], axis=1)


def _tc_matmul_chunks(x, w):
    def body(x_ref, w_ref, out_ref):
        xw = jnp.dot(x_ref[...], w_ref[...],
                     preferred_element_type=jnp.float32)
        for c in range(_CC):
            out_ref[c] = xw[:, c * _CW:(c + 1) * _CW]
    return _PCALL(
        body,
        grid=(_NBLK,),
        in_specs=[pl.BlockSpec((_ROWB, _P), lambda r: (r, 0)),
                  pl.BlockSpec((_P, _P), lambda r: (0, 0))],
        out_specs=pl.BlockSpec((_CC, _ROWB, _CW), lambda r: (0, r, 0)),
        out_shape=jax.ShapeDtypeStruct((_CC, _NP, _CW), jnp.float32),
    )(x, w)


def _tc_c1(xs4, adjT, wi1, wi2):
    def body(xs_ref, adjt_ref, wi1_ref, wi2_ref, H1_ref, h1_ref):
        xsb = _cat4(xs_ref)
        t = jnp.dot(xsb, wi1_ref[...], preferred_element_type=jnp.float32)
        t = jnp.maximum(t + xsb, 0.0)
        logits = jnp.dot(t, wi2_ref[...], preferred_element_type=jnp.float32)
        cols = lax.broadcasted_iota(jnp.int32, logits.shape, 1)
        logits = jnp.where(cols < _EMB, logits, -1e30)
        m = jnp.max(logits, axis=1, keepdims=True)
        p = jnp.exp(logits - m)
        H1 = p / jnp.sum(p, axis=1, keepdims=True)
        H1_ref[...] = H1
        g = H1 * adjt_ref[...]
        s = jnp.sum(g, axis=1, keepdims=True)
        gn = g / (s + 1e-8)
        part = lax.dot_general(gn, xsb, (((0,), (0,)), ((), ())),
                               preferred_element_type=jnp.float32)

        @pl.when(pl.program_id(0) == 0)
        def _init():
            h1_ref[...] = jnp.zeros_like(h1_ref)
        h1_ref[...] += part

    return _PCALL(
        body,
        grid=(_NBLK,),
        in_specs=[pl.BlockSpec((_CC, _ROWB, _CW), lambda r: (0, r, 0)),
                  pl.BlockSpec((_ROWB, _P), lambda r: (r, 0)),
                  pl.BlockSpec((_P, _P), lambda r: (0, 0)),
                  pl.BlockSpec((_P, _P), lambda r: (0, 0))],
        out_specs=[pl.BlockSpec((_ROWB, _P), lambda r: (r, 0)),
                   pl.BlockSpec((_P, _P), lambda r: (0, 0))],
        out_shape=[jax.ShapeDtypeStruct((_N, _P), jnp.float32),
                   jax.ShapeDtypeStruct((_P, _P), jnp.float32)],
    )(xs4, adjT, wi1, wi2)


def _nrm(v):
    n = jnp.sqrt(jnp.sum(v * v, axis=1, keepdims=True))
    return v / jnp.maximum(n, 1e-12)


def _tc_c2_fused(H1, h1, xs4, acci, acch, wnext):
    def body(H1_ref, h1_ref, xs_ref, acci_ref, acch_ref, wn_ref,
             xw_ref, accio_ref, accho_ref):
        xsb = _cat4(xs_ref)
        h2 = jnp.dot(H1_ref[...], h1_ref[...],
                     preferred_element_type=jnp.float32)
        xo = h2 + xsb
        accio_ref[...] = acci_ref[...] + _nrm(xo)
        accho_ref[...] = acch_ref[...] + _nrm(h2)
        xw = jnp.dot(xo, wn_ref[...], preferred_element_type=jnp.float32)
        for c in range(_CC):
            xw_ref[c] = xw[:, c * _CW:(c + 1) * _CW]

    return _PCALL(
        body,
        grid=(_NBLK,),
        in_specs=[pl.BlockSpec((_ROWB, _P), lambda r: (r, 0)),
                  pl.BlockSpec((_P, _P), lambda r: (0, 0)),
                  pl.BlockSpec((_CC, _ROWB, _CW), lambda r: (0, r, 0)),
                  pl.BlockSpec((_ROWB, _P), lambda r: (r, 0)),
                  pl.BlockSpec((_ROWB, _P), lambda r: (r, 0)),
                  pl.BlockSpec((_P, _P), lambda r: (0, 0))],
        out_specs=[pl.BlockSpec((_CC, _ROWB, _CW), lambda r: (0, r, 0)),
                   pl.BlockSpec((_ROWB, _P), lambda r: (r, 0)),
                   pl.BlockSpec((_ROWB, _P), lambda r: (r, 0))],
        out_shape=[jax.ShapeDtypeStruct((_CC, _NP, _CW), jnp.float32),
                   jax.ShapeDtypeStruct((_N, _P), jnp.float32),
                   jax.ShapeDtypeStruct((_N, _P), jnp.float32)],
    )(H1, h1, xs4, acci, acch, wnext)


def _tc_c2_final(H1, h1, xs4, acci, acch):
    def body(H1_ref, h1_ref, xs_ref, acci_ref, acch_ref,
             item_ref, hs_ref):
        xsb = _cat4(xs_ref)
        h2 = jnp.dot(H1_ref[...], h1_ref[...],
                     preferred_element_type=jnp.float32)
        xo = h2 + xsb
        item_ref[...] = (acci_ref[...] + _nrm(xo)) * (1.0 / 3.0)
        hs_ref[...] = (acch_ref[...] + _nrm(h2)) * 0.5

    return _PCALL(
        body,
        grid=(_NBLK,),
        in_specs=[pl.BlockSpec((_ROWB, _P), lambda r: (r, 0)),
                  pl.BlockSpec((_P, _P), lambda r: (0, 0)),
                  pl.BlockSpec((_CC, _ROWB, _CW), lambda r: (0, r, 0)),
                  pl.BlockSpec((_ROWB, _P), lambda r: (r, 0)),
                  pl.BlockSpec((_ROWB, _P), lambda r: (r, 0))],
        out_specs=[pl.BlockSpec((_ROWB, _P), lambda r: (r, 0)),
                   pl.BlockSpec((_ROWB, _P), lambda r: (r, 0))],
        out_shape=[jax.ShapeDtypeStruct((_N, _P), jnp.float32),
                   jax.ShapeDtypeStruct((_N, _P), jnp.float32)],
    )(H1, h1, xs4, acci, acch)


def _pad2(w):
    return jnp.pad(w.astype(jnp.float32),
                   ((0, _P - w.shape[0]), (0, _P - w.shape[1])))


def kernel(adj, edge_index, edge_val, embedding, channel,
           W_item0, W_item1, W_i1, W_i2):
    del channel
    f32 = jnp.float32
    emb = jnp.pad(embedding.astype(f32), ((0, 0), (0, _P - _EMB)))
    W0p, W1p = _pad2(W_item0), _pad2(W_item1)
    Wi1p, Wi2p = _pad2(W_i1), _pad2(W_i2)
    adjT = jnp.pad(adj.T.astype(f32), ((0, 0), (0, _P - adj.shape[0])))

    rowp = jnp.pad(edge_index[0], (0, _EPAD - _E)).reshape(_R128, 128)
    colp = jnp.pad(edge_index[1], (0, _EPAD - _E))
    valp = jnp.pad(edge_val.astype(f32), (0, _EPAD - _E))

    acci = emb
    acch = jnp.zeros_like(emb)
    xw = _tc_matmul_chunks(emb, W0p)
    item = hs = None
    for i in range(2):
        xs4 = _sc_edge_segsum(xw.reshape(_CC * _NP, _CW), colp, rowp, valp)
        xs4 = xs4.reshape(_CC, _NP, _CW)
        H1, h1 = _tc_c1(xs4, adjT, Wi1p, Wi2p)
        if i == 0:
            xw, acci, acch = _tc_c2_fused(H1, h1, xs4, acci, acch, W1p)
        else:
            item, hs = _tc_c2_final(H1, h1, xs4, acci, acch)
    return item[:, :_EMB], hs[:, :_EMB]


# speedup vs baseline: 5.3135x; 1.3805x over previous
import functools

import jax
import jax.numpy as jnp
from jax import lax
from jax.experimental import pallas as pl
from jax.experimental.pallas import tpu as pltpu
from jax.experimental.pallas import tpu_sc as plsc

_PCALL = pl.pallas_call

_N = 50000
_E = 800000
_EMB = 100
_P = 128
_CC = 4
_CW = 32
_NC = 2
_NS = 16
_NP = 51200
_EPAD = 802816
_R128 = _EPAD // 128
_TILE_E = _EPAD // _NS
_SUP = 256
_GRP = 1024
_NSUP = _TILE_E // _SUP
_NGRP = _TILE_E // _GRP
_TILE_R8 = _TILE_E // 128
_ROWS_T = _NP // _NS
_ZB = 160
_NZ = _ROWS_T // _ZB
_ROWB = 2000
_NBLK = _N // _ROWB


def _sc_edge_segsum(table, col1d, row2d, val1d):
    mesh = plsc.VectorSubcoreMesh(
        core_axis_name="c", subcore_axis_name="s",
        num_cores=_NC, num_subcores=_NS)

    @functools.partial(
        pl.kernel,
        out_type=jax.ShapeDtypeStruct((_CC * _NP, _CW), jnp.float32),
        mesh=mesh,
        compiler_params=pltpu.CompilerParams(use_tc_tiling_on_sc=False),
        scratch_types=[
            [pltpu.VMEM((_GRP,), jnp.int32)] * 2,
            [pltpu.VMEM((8, 128), jnp.int32)] * 2,
            [pltpu.VMEM((_GRP,), jnp.float32)] * 2,
            [pltpu.VMEM((_SUP, _CW), jnp.float32)] * 2,
            pltpu.VMEM((_ZB, _CW), jnp.float32),
            pltpu.VMEM_SHARED((_NP, _CW), jnp.float32),
            [pltpu.SemaphoreType.DMA] * 2,
            pltpu.SemaphoreType.DMA,
        ],
    )
    def k(table_h, col_h, row_h, val_h, out_h, colv, rowv, valv, rows, zb,
          acc, semg, semi):
        core = lax.axis_index("c")
        sub = lax.axis_index("s")
        zvec = jnp.zeros((16,), jnp.float32)

        def zfill(r, _):
            zb[r, pl.ds(0, 16)] = zvec
            zb[r, pl.ds(16, 16)] = zvec
            return 0
        lax.fori_loop(0, _ZB, zfill, 0)

        def add_cc(colv_b, ccvec):
            for k16 in range(_GRP // 16):
                colv_b[pl.ds(k16 * 16, 16)] = (
                    colv_b[pl.ds(k16 * 16, 16)] + ccvec)

        def fire_gather(p, q, off):
            for a in range(2):
                pltpu.async_copy(
                    table_h.at[colv[q].at[pl.ds(off + a * 128, 128)]],
                    rows[p].at[pl.ds(a * 128, 128)], semg[p])

        def consume(p, q, w):
            pltpu.make_async_copy(table_h.at[pl.ds(0, _SUP)], rows[p],
                                  semg[p]).wait()

            def scale(k16, _):
                v16 = valv[q][pl.ds(w * _SUP + k16 * 16, 16)]
                for l in range(16):
                    e = k16 * 16 + l
                    b = jnp.full((16,), v16[l], jnp.float32)
                    rows[p][e, pl.ds(0, 16)] = rows[p][e, pl.ds(0, 16)] * b
                    rows[p][e, pl.ds(16, 16)] = rows[p][e, pl.ds(16, 16)] * b
                return 0
            lax.fori_loop(0, _SUP // 16, scale, 0)

            for a in range(2):
                pltpu.sync_copy(rows[p].at[pl.ds(a * 128, 128)],
                                acc.at[rowv[q].at[w * 2 + a]], add=True)

        for j in range(_CC // _NC):
            cc = core * (_CC // _NC) + j
            for z in range(_NZ):
                pltpu.sync_copy(zb, acc.at[pl.ds(sub * _ROWS_T + z * _ZB, _ZB)])
            plsc.subcore_barrier()

            ccvec = jnp.full((16,), cc * _NP, jnp.int32)
            ebase0 = sub * _TILE_E
            rbase0 = sub * _TILE_R8

            pltpu.sync_copy(col_h.at[pl.ds(ebase0, _GRP)], colv[0])
            pltpu.sync_copy(val_h.at[pl.ds(ebase0, _GRP)], valv[0])
            pltpu.sync_copy(row_h.at[pl.ds(rbase0, 8)], rowv[0])
            add_cc(colv[0], ccvec)
            fire_gather(0, 0, 0)

            def body(sb, _):
                w = sb % 4
                g = sb // 4

                for q in range(2):
                    cond = ((w == 3) & (sb < (_NSUP - 1))
                            & (((g + 1) % 2) == q))

                    def _drain_idx(q=q):
                        pltpu.make_async_copy(col_h.at[pl.ds(0, _GRP)],
                                              colv[q], semi).wait()
                        pltpu.make_async_copy(val_h.at[pl.ds(0, _GRP)],
                                              valv[q], semi).wait()
                        pltpu.make_async_copy(row_h.at[pl.ds(0, 8)],
                                              rowv[q], semi).wait()
                        add_cc(colv[q], ccvec)
                    pl.when(cond)(_drain_idx)

                for p in range(2):
                    for q in range(2):
                        cond = ((sb < (_NSUP - 1))
                                & (((sb + 1) % 2) == p)
                                & ((((sb + 1) // 4) % 2) == q))
                        off = ((sb + 1) % 4) * _SUP
                        pl.when(cond)(functools.partial(fire_gather, p, q, off))

                for q in range(2):
                    cond = (w == 0) & (sb < (_NSUP - 4)) & (((g + 1) % 2) == q)

                    def _fire_idx(q=q):
                        eb = ebase0 + (g + 1) * _GRP
                        pltpu.async_copy(col_h.at[pl.ds(eb, _GRP)],
                                         colv[q], semi)
                        pltpu.async_copy(val_h.at[pl.ds(eb, _GRP)],
                                         valv[q], semi)
                        rb = rbase0 + (g + 1) * 8
                        pltpu.async_copy(row_h.at[pl.ds(rb, 8)], rowv[q], semi)
                    pl.when(cond)(_fire_idx)

                for p in range(2):
                    for q in range(2):
                        pl.when(((sb % 2) == p) & ((g % 2) == q))(
                            functools.partial(consume, p, q, w))
                return 0
            lax.fori_loop(0, _NSUP, body, 0)
            plsc.subcore_barrier()

            obase = cc * _NP + sub * _ROWS_T
            for z in range(_NZ):
                pltpu.sync_copy(acc.at[pl.ds(sub * _ROWS_T + z * _ZB, _ZB)],
                                out_h.at[pl.ds(obase + z * _ZB, _ZB)])

    return k(table, col1d, row2d, val1d)


def _cat4(xs_ref):
    return jnp.concatenate([xs_ref[c] for c in range(_CC)], axis=1)


def _tc_matmul_chunks(x, w):
    def body(x_ref, w_ref, out_ref):
        xw = jnp.dot(x_ref[...], w_ref[...],
                     preferred_element_type=jnp.float32)
        for c in range(_CC):
            out_ref[c] = xw[:, c * _CW:(c + 1) * _CW]
    return _PCALL(
        body,
        grid=(_NBLK,),
        in_specs=[pl.BlockSpec((_ROWB, _P), lambda r: (r, 0)),
                  pl.BlockSpec((_P, _P), lambda r: (0, 0))],
        out_specs=pl.BlockSpec((_CC, _ROWB, _CW), lambda r: (0, r, 0)),
        out_shape=jax.ShapeDtypeStruct((_CC, _NP, _CW), jnp.float32),
    )(x, w)


def _tc_c1(xs4, adjT, wi1, wi2):
    def body(xs_ref, adjt_ref, wi1_ref, wi2_ref, H1_ref, h1_ref):
        xsb = _cat4(xs_ref)
        t = jnp.dot(xsb, wi1_ref[...], preferred_element_type=jnp.float32)
        t = jnp.maximum(t + xsb, 0.0)
        logits = jnp.dot(t, wi2_ref[...], preferred_element_type=jnp.float32)
        cols = lax.broadcasted_iota(jnp.int32, logits.shape, 1)
        logits = jnp.where(cols < _EMB, logits, -1e30)
        m = jnp.max(logits, axis=1, keepdims=True)
        p = jnp.exp(logits - m)
        H1 = p / jnp.sum(p, axis=1, keepdims=True)
        H1_ref[...] = H1
        g = H1 * adjt_ref[...]
        s = jnp.sum(g, axis=1, keepdims=True)
        gn = g / (s + 1e-8)
        part = lax.dot_general(gn, xsb, (((0,), (0,)), ((), ())),
                               preferred_element_type=jnp.float32)

        @pl.when(pl.program_id(0) == 0)
        def _init():
            h1_ref[...] = jnp.zeros_like(h1_ref)
        h1_ref[...] += part

    return _PCALL(
        body,
        grid=(_NBLK,),
        in_specs=[pl.BlockSpec((_CC, _ROWB, _CW), lambda r: (0, r, 0)),
                  pl.BlockSpec((_ROWB, _P), lambda r: (r, 0)),
                  pl.BlockSpec((_P, _P), lambda r: (0, 0)),
                  pl.BlockSpec((_P, _P), lambda r: (0, 0))],
        out_specs=[pl.BlockSpec((_ROWB, _P), lambda r: (r, 0)),
                   pl.BlockSpec((_P, _P), lambda r: (0, 0))],
        out_shape=[jax.ShapeDtypeStruct((_N, _P), jnp.float32),
                   jax.ShapeDtypeStruct((_P, _P), jnp.float32)],
    )(xs4, adjT, wi1, wi2)


def _nrm(v):
    n = jnp.sqrt(jnp.sum(v * v, axis=1, keepdims=True))
    return v / jnp.maximum(n, 1e-12)


def _tc_c2_fused(H1, h1, xs4, acci, acch, wnext):
    def body(H1_ref, h1_ref, xs_ref, acci_ref, acch_ref, wn_ref,
             xw_ref, accio_ref, accho_ref):
        xsb = _cat4(xs_ref)
        h2 = jnp.dot(H1_ref[...], h1_ref[...],
                     preferred_element_type=jnp.float32)
        xo = h2 + xsb
        accio_ref[...] = acci_ref[...] + _nrm(xo)
        accho_ref[...] = acch_ref[...] + _nrm(h2)
        xw = jnp.dot(xo, wn_ref[...], preferred_element_type=jnp.float32)
        for c in range(_CC):
            xw_ref[c] = xw[:, c * _CW:(c + 1) * _CW]

    return _PCALL(
        body,
        grid=(_NBLK,),
        in_specs=[pl.BlockSpec((_ROWB, _P), lambda r: (r, 0)),
                  pl.BlockSpec((_P, _P), lambda r: (0, 0)),
                  pl.BlockSpec((_CC, _ROWB, _CW), lambda r: (0, r, 0)),
                  pl.BlockSpec((_ROWB, _P), lambda r: (r, 0)),
                  pl.BlockSpec((_ROWB, _P), lambda r: (r, 0)),
                  pl.BlockSpec((_P, _P), lambda r: (0, 0))],
        out_specs=[pl.BlockSpec((_CC, _ROWB, _CW), lambda r: (0, r, 0)),
                   pl.BlockSpec((_ROWB, _P), lambda r: (r, 0)),
                   pl.BlockSpec((_ROWB, _P), lambda r: (r, 0))],
        out_shape=[jax.ShapeDtypeStruct((_CC, _NP, _CW), jnp.float32),
                   jax.ShapeDtypeStruct((_N, _P), jnp.float32),
                   jax.ShapeDtypeStruct((_N, _P), jnp.float32)],
    )(H1, h1, xs4, acci, acch, wnext)


def _tc_c2_final(H1, h1, xs4, acci, acch):
    def body(H1_ref, h1_ref, xs_ref, acci_ref, acch_ref,
             item_ref, hs_ref):
        xsb = _cat4(xs_ref)
        h2 = jnp.dot(H1_ref[...], h1_ref[...],
                     preferred_element_type=jnp.float32)
        xo = h2 + xsb
        item_ref[...] = (acci_ref[...] + _nrm(xo)) * (1.0 / 3.0)
        hs_ref[...] = (acch_ref[...] + _nrm(h2)) * 0.5

    return _PCALL(
        body,
        grid=(_NBLK,),
        in_specs=[pl.BlockSpec((_ROWB, _P), lambda r: (r, 0)),
                  pl.BlockSpec((_P, _P), lambda r: (0, 0)),
                  pl.BlockSpec((_CC, _ROWB, _CW), lambda r: (0, r, 0)),
                  pl.BlockSpec((_ROWB, _P), lambda r: (r, 0)),
                  pl.BlockSpec((_ROWB, _P), lambda r: (r, 0))],
        out_specs=[pl.BlockSpec((_ROWB, _P), lambda r: (r, 0)),
                   pl.BlockSpec((_ROWB, _P), lambda r: (r, 0))],
        out_shape=[jax.ShapeDtypeStruct((_N, _P), jnp.float32),
                   jax.ShapeDtypeStruct((_N, _P), jnp.float32)],
    )(H1, h1, xs4, acci, acch)


def _pad2(w):
    return jnp.pad(w.astype(jnp.float32),
                   ((0, _P - w.shape[0]), (0, _P - w.shape[1])))


def kernel(adj, edge_index, edge_val, embedding, channel,
           W_item0, W_item1, W_i1, W_i2):
    del channel
    f32 = jnp.float32
    emb = jnp.pad(embedding.astype(f32), ((0, 0), (0, _P - _EMB)))
    W0p, W1p = _pad2(W_item0), _pad2(W_item1)
    Wi1p, Wi2p = _pad2(W_i1), _pad2(W_i2)
    adjT = jnp.pad(adj.T.astype(f32), ((0, 0), (0, _P - adj.shape[0])))

    rowp = jnp.pad(edge_index[0], (0, _EPAD - _E)).reshape(_R128, 128)
    colp = jnp.pad(edge_index[1], (0, _EPAD - _E))
    valp = jnp.pad(edge_val.astype(f32), (0, _EPAD - _E))

    acci = emb
    acch = jnp.zeros_like(emb)
    xw = _tc_matmul_chunks(emb, W0p)
    item = hs = None
    for i in range(2):
        xs4 = _sc_edge_segsum(xw.reshape(_CC * _NP, _CW), colp, rowp, valp)
        xs4 = xs4.reshape(_CC, _NP, _CW)
        H1, h1 = _tc_c1(xs4, adjT, Wi1p, Wi2p)
        if i == 0:
            xw, acci, acch = _tc_c2_fused(H1, h1, xs4, acci, acch, W1p)
        else:
            item, hs = _tc_c2_final(H1, h1, xs4, acci, acch)
    return item[:, :_EMB], hs[:, :_EMB]


# speedup vs baseline: 5.4247x; 1.0209x over previous
import functools

import jax
import jax.numpy as jnp
from jax import lax
from jax.experimental import pallas as pl
from jax.experimental.pallas import tpu as pltpu
from jax.experimental.pallas import tpu_sc as plsc

_PCALL = pl.pallas_call

_N = 50000
_E = 800000
_EMB = 100
_P = 128
_CC = 4
_CW = 32
_NC = 2
_NS = 16
_NP = 51200
_EPAD = 802816
_R128 = _EPAD // 128
_TILE_E = _EPAD // _NS
_SUP = 256
_GRP = 1024
_NSUP = _TILE_E // _SUP
_NGRP = _TILE_E // _GRP
_TILE_R8 = _TILE_E // 128
_ROWS_T = _NP // _NS
_ZB = 160
_NZ = _ROWS_T // _ZB
_ROWB = 2000
_NBLK = _N // _ROWB


def _sc_edge_segsum(table, col1d, row2d, val1d):
    mesh = plsc.VectorSubcoreMesh(
        core_axis_name="c", subcore_axis_name="s",
        num_cores=_NC, num_subcores=_NS)

    @functools.partial(
        pl.kernel,
        out_type=jax.ShapeDtypeStruct((_CC * _NP, _CW), jnp.float32),
        mesh=mesh,
        compiler_params=pltpu.CompilerParams(use_tc_tiling_on_sc=False),
        scratch_types=[
            [pltpu.VMEM((_GRP,), jnp.int32)] * 2,
            [pltpu.VMEM((8, 128), jnp.int32)] * 2,
            [pltpu.VMEM((_GRP,), jnp.float32)] * 2,
            [pltpu.VMEM((_SUP, _CW), jnp.float32)] * 2,
            pltpu.VMEM((_ZB, _CW), jnp.float32),
            pltpu.VMEM_SHARED((_NP, _CW), jnp.float32),
            [pltpu.SemaphoreType.DMA] * 2,
            [pltpu.SemaphoreType.DMA] * 2,
            pltpu.SemaphoreType.DMA,
        ],
    )
    def k(table_h, col_h, row_h, val_h, out_h, colv, rowv, valv, rows, zb,
          acc, semg, semsc, semi):
        core = lax.axis_index("c")
        sub = lax.axis_index("s")
        zvec = jnp.zeros((16,), jnp.float32)

        def zfill(r, _):
            zb[r, pl.ds(0, 16)] = zvec
            zb[r, pl.ds(16, 16)] = zvec
            return 0
        lax.fori_loop(0, _ZB, zfill, 0)

        def add_cc(colv_b, ccvec):
            for k16 in range(_GRP // 16):
                colv_b[pl.ds(k16 * 16, 16)] = (
                    colv_b[pl.ds(k16 * 16, 16)] + ccvec)

        def fire_gather(p, q, off):
            for a in range(2):
                pltpu.async_copy(
                    table_h.at[colv[q].at[pl.ds(off + a * 128, 128)]],
                    rows[p].at[pl.ds(a * 128, 128)], semg[p])

        def consume(p, q, w):
            pltpu.make_async_copy(table_h.at[pl.ds(0, _SUP)], rows[p],
                                  semg[p]).wait()

            def scale(k16, _):
                v16 = valv[q][pl.ds(w * _SUP + k16 * 16, 16)]
                for l in range(16):
                    e = k16 * 16 + l
                    b = jnp.full((16,), v16[l], jnp.float32)
                    rows[p][e, pl.ds(0, 16)] = rows[p][e, pl.ds(0, 16)] * b
                    rows[p][e, pl.ds(16, 16)] = rows[p][e, pl.ds(16, 16)] * b
                return 0
            lax.fori_loop(0, _SUP // 16, scale, 0)

            for a in range(2):
                pltpu.async_copy(rows[p].at[pl.ds(a * 128, 128)],
                                 acc.at[rowv[q].at[w * 2 + a]], semsc[p],
                                 add=True)

        for j in range(_CC // _NC):
            cc = core * (_CC // _NC) + j
            for z in range(_NZ):
                pltpu.sync_copy(zb, acc.at[pl.ds(sub * _ROWS_T + z * _ZB, _ZB)])
            plsc.subcore_barrier()

            ccvec = jnp.full((16,), cc * _NP, jnp.int32)
            ebase0 = sub * _TILE_E
            rbase0 = sub * _TILE_R8

            pltpu.sync_copy(col_h.at[pl.ds(ebase0, _GRP)], colv[0])
            pltpu.sync_copy(val_h.at[pl.ds(ebase0, _GRP)], valv[0])
            pltpu.sync_copy(row_h.at[pl.ds(rbase0, 8)], rowv[0])
            add_cc(colv[0], ccvec)
            fire_gather(0, 0, 0)

            def body(sb, _):
                w = sb % 4
                g = sb // 4

                for q in range(2):
                    cond = ((w == 3) & (sb < (_NSUP - 1))
                            & (((g + 1) % 2) == q))

                    def _drain_idx(q=q):
                        pltpu.make_async_copy(col_h.at[pl.ds(0, _GRP)],
                                              colv[q], semi).wait()
                        pltpu.make_async_copy(val_h.at[pl.ds(0, _GRP)],
                                              valv[q], semi).wait()
                        pltpu.make_async_copy(row_h.at[pl.ds(0, 8)],
                                              rowv[q], semi).wait()
                        add_cc(colv[q], ccvec)
                    pl.when(cond)(_drain_idx)

                for p in range(2):
                    cond = ((sb < (_NSUP - 1)) & (sb >= 1)
                            & (((sb + 1) % 2) == p))

                    def _drain_sc(p=p):
                        pltpu.make_async_copy(table_h.at[pl.ds(0, _SUP)],
                                              rows[p], semsc[p]).wait()
                    pl.when(cond)(_drain_sc)

                for p in range(2):
                    for q in range(2):
                        cond = ((sb < (_NSUP - 1))
                                & (((sb + 1) % 2) == p)
                                & ((((sb + 1) // 4) % 2) == q))
                        off = ((sb + 1) % 4) * _SUP
                        pl.when(cond)(functools.partial(fire_gather, p, q, off))

                for q in range(2):
                    cond = (w == 0) & (sb < (_NSUP - 4)) & (((g + 1) % 2) == q)

                    def _fire_idx(q=q):
                        eb = ebase0 + (g + 1) * _GRP
                        pltpu.async_copy(col_h.at[pl.ds(eb, _GRP)],
                                         colv[q], semi)
                        pltpu.async_copy(val_h.at[pl.ds(eb, _GRP)],
                                         valv[q], semi)
                        rb = rbase0 + (g + 1) * 8
                        pltpu.async_copy(row_h.at[pl.ds(rb, 8)], rowv[q], semi)
                    pl.when(cond)(_fire_idx)

                for p in range(2):
                    for q in range(2):
                        pl.when(((sb % 2) == p) & ((g % 2) == q))(
                            functools.partial(consume, p, q, w))
                return 0
            lax.fori_loop(0, _NSUP, body, 0)
            for p in range(2):
                pltpu.make_async_copy(table_h.at[pl.ds(0, _SUP)],
                                      rows[p], semsc[p]).wait()
            plsc.subcore_barrier()

            obase = cc * _NP + sub * _ROWS_T
            for z in range(_NZ):
                pltpu.sync_copy(acc.at[pl.ds(sub * _ROWS_T + z * _ZB, _ZB)],
                                out_h.at[pl.ds(obase + z * _ZB, _ZB)])

    return k(table, col1d, row2d, val1d)


def _cat4(xs_ref):
    return jnp.concatenate([xs_ref[c] for c in range(_CC)], axis=1)


def _tc_matmul_chunks(x, w):
    def body(x_ref, w_ref, out_ref):
        xw = jnp.dot(x_ref[...], w_ref[...],
                     preferred_element_type=jnp.float32)
        for c in range(_CC):
            out_ref[c] = xw[:, c * _CW:(c + 1) * _CW]
    return _PCALL(
        body,
        grid=(_NBLK,),
        in_specs=[pl.BlockSpec((_ROWB, _P), lambda r: (r, 0)),
                  pl.BlockSpec((_P, _P), lambda r: (0, 0))],
        out_specs=pl.BlockSpec((_CC, _ROWB, _CW), lambda r: (0, r, 0)),
        out_shape=jax.ShapeDtypeStruct((_CC, _NP, _CW), jnp.float32),
    )(x, w)


def _tc_c1(xs4, adjT, wi1, wi2):
    def body(xs_ref, adjt_ref, wi1_ref, wi2_ref, H1_ref, h1_ref):
        xsb = _cat4(xs_ref)
        t = jnp.dot(xsb, wi1_ref[...], preferred_element_type=jnp.float32)
        t = jnp.maximum(t + xsb, 0.0)
        logits = jnp.dot(t, wi2_ref[...], preferred_element_type=jnp.float32)
        cols = lax.broadcasted_iota(jnp.int32, logits.shape, 1)
        logits = jnp.where(cols < _EMB, logits, -1e30)
        m = jnp.max(logits, axis=1, keepdims=True)
        p = jnp.exp(logits - m)
        H1 = p / jnp.sum(p, axis=1, keepdims=True)
        H1_ref[...] = H1
        g = H1 * adjt_ref[...]
        s = jnp.sum(g, axis=1, keepdims=True)
        gn = g / (s + 1e-8)
        part = lax.dot_general(gn, xsb, (((0,), (0,)), ((), ())),
                               preferred_element_type=jnp.float32)

        @pl.when(pl.program_id(0) == 0)
        def _init():
            h1_ref[...] = jnp.zeros_like(h1_ref)
        h1_ref[...] += part

    return _PCALL(
        body,
        grid=(_NBLK,),
        in_specs=[pl.BlockSpec((_CC, _ROWB, _CW), lambda r: (0, r, 0)),
                  pl.BlockSpec((_ROWB, _P), lambda r: (r, 0)),
                  pl.BlockSpec((_P, _P), lambda r: (0, 0)),
                  pl.BlockSpec((_P, _P), lambda r: (0, 0))],
        out_specs=[pl.BlockSpec((_ROWB, _P), lambda r: (r, 0)),
                   pl.BlockSpec((_P, _P), lambda r: (0, 0))],
        out_shape=[jax.ShapeDtypeStruct((_N, _P), jnp.float32),
                   jax.ShapeDtypeStruct((_P, _P), jnp.float32)],
    )(xs4, adjT, wi1, wi2)


def _nrm(v):
    n = jnp.sqrt(jnp.sum(v * v, axis=1, keepdims=True))
    return v / jnp.maximum(n, 1e-12)


def _tc_c2_fused(H1, h1, xs4, acci, acch, wnext):
    def body(H1_ref, h1_ref, xs_ref, acci_ref, acch_ref, wn_ref,
             xw_ref, accio_ref, accho_ref):
        xsb = _cat4(xs_ref)
        h2 = jnp.dot(H1_ref[...], h1_ref[...],
                     preferred_element_type=jnp.float32)
        xo = h2 + xsb
        accio_ref[...] = acci_ref[...] + _nrm(xo)
        accho_ref[...] = acch_ref[...] + _nrm(h2)
        xw = jnp.dot(xo, wn_ref[...], preferred_element_type=jnp.float32)
        for c in range(_CC):
            xw_ref[c] = xw[:, c * _CW:(c + 1) * _CW]

    return _PCALL(
        body,
        grid=(_NBLK,),
        in_specs=[pl.BlockSpec((_ROWB, _P), lambda r: (r, 0)),
                  pl.BlockSpec((_P, _P), lambda r: (0, 0)),
                  pl.BlockSpec((_CC, _ROWB, _CW), lambda r: (0, r, 0)),
                  pl.BlockSpec((_ROWB, _P), lambda r: (r, 0)),
                  pl.BlockSpec((_ROWB, _P), lambda r: (r, 0)),
                  pl.BlockSpec((_P, _P), lambda r: (0, 0))],
        out_specs=[pl.BlockSpec((_CC, _ROWB, _CW), lambda r: (0, r, 0)),
                   pl.BlockSpec((_ROWB, _P), lambda r: (r, 0)),
                   pl.BlockSpec((_ROWB, _P), lambda r: (r, 0))],
        out_shape=[jax.ShapeDtypeStruct((_CC, _NP, _CW), jnp.float32),
                   jax.ShapeDtypeStruct((_N, _P), jnp.float32),
                   jax.ShapeDtypeStruct((_N, _P), jnp.float32)],
    )(H1, h1, xs4, acci, acch, wnext)


def _tc_c2_final(H1, h1, xs4, acci, acch):
    def body(H1_ref, h1_ref, xs_ref, acci_ref, acch_ref,
             item_ref, hs_ref):
        xsb = _cat4(xs_ref)
        h2 = jnp.dot(H1_ref[...], h1_ref[...],
                     preferred_element_type=jnp.float32)
        xo = h2 + xsb
        item_ref[...] = (acci_ref[...] + _nrm(xo)) * (1.0 / 3.0)
        hs_ref[...] = (acch_ref[...] + _nrm(h2)) * 0.5

    return _PCALL(
        body,
        grid=(_NBLK,),
        in_specs=[pl.BlockSpec((_ROWB, _P), lambda r: (r, 0)),
                  pl.BlockSpec((_P, _P), lambda r: (0, 0)),
                  pl.BlockSpec((_CC, _ROWB, _CW), lambda r: (0, r, 0)),
                  pl.BlockSpec((_ROWB, _P), lambda r: (r, 0)),
                  pl.BlockSpec((_ROWB, _P), lambda r: (r, 0))],
        out_specs=[pl.BlockSpec((_ROWB, _P), lambda r: (r, 0)),
                   pl.BlockSpec((_ROWB, _P), lambda r: (r, 0))],
        out_shape=[jax.ShapeDtypeStruct((_N, _P), jnp.float32),
                   jax.ShapeDtypeStruct((_N, _P), jnp.float32)],
    )(H1, h1, xs4, acci, acch)


def _pad2(w):
    return jnp.pad(w.astype(jnp.float32),
                   ((0, _P - w.shape[0]), (0, _P - w.shape[1])))


def kernel(adj, edge_index, edge_val, embedding, channel,
           W_item0, W_item1, W_i1, W_i2):
    del channel
    f32 = jnp.float32
    emb = jnp.pad(embedding.astype(f32), ((0, 0), (0, _P - _EMB)))
    W0p, W1p = _pad2(W_item0), _pad2(W_item1)
    Wi1p, Wi2p = _pad2(W_i1), _pad2(W_i2)
    adjT = jnp.pad(adj.T.astype(f32), ((0, 0), (0, _P - adj.shape[0])))

    rowp = jnp.pad(edge_index[0], (0, _EPAD - _E)).reshape(_R128, 128)
    colp = jnp.pad(edge_index[1], (0, _EPAD - _E))
    valp = jnp.pad(edge_val.astype(f32), (0, _EPAD - _E))

    acci = emb
    acch = jnp.zeros_like(emb)
    xw = _tc_matmul_chunks(emb, W0p)
    item = hs = None
    for i in range(2):
        xs4 = _sc_edge_segsum(xw.reshape(_CC * _NP, _CW), colp, rowp, valp)
        xs4 = xs4.reshape(_CC, _NP, _CW)
        H1, h1 = _tc_c1(xs4, adjT, Wi1p, Wi2p)
        if i == 0:
            xw, acci, acch = _tc_c2_fused(H1, h1, xs4, acci, acch, W1p)
        else:
            item, hs = _tc_c2_final(H1, h1, xs4, acci, acch)
    return item[:, :_EMB], hs[:, :_EMB]


# speedup vs baseline: 5.6388x; 1.0395x over previous
import functools

import jax
import jax.numpy as jnp
from jax import lax
from jax.experimental import pallas as pl
from jax.experimental.pallas import tpu as pltpu
from jax.experimental.pallas import tpu_sc as plsc

_PCALL = pl.pallas_call

_N = 50000
_E = 800000
_EMB = 100
_P = 128
_CC = 4
_CW = 32
_NC = 2
_NS = 16
_NP = 51200
_EPAD = 802816
_R128 = _EPAD // 128
_TILE_E = _EPAD // _NS
_SUP = 256
_GRP = 1024
_NSUP = _TILE_E // _SUP
_NGRP = _TILE_E // _GRP
_TILE_R8 = _TILE_E // 128
_ROWS_T = _NP // _NS
_ZB = 160
_NZ = _ROWS_T // _ZB
_ROWB = 2000
_NBLK = _N // _ROWB


def _sc_edge_segsum(table, col1d, row2d, val1d):
    mesh = plsc.VectorSubcoreMesh(
        core_axis_name="c", subcore_axis_name="s",
        num_cores=_NC, num_subcores=_NS)

    @functools.partial(
        pl.kernel,
        out_type=jax.ShapeDtypeStruct((_CC * _NP, _CW), jnp.float32),
        mesh=mesh,
        compiler_params=pltpu.CompilerParams(use_tc_tiling_on_sc=False),
        scratch_types=[
            [pltpu.VMEM((_GRP,), jnp.int32)] * 2,
            [pltpu.VMEM((8, 128), jnp.int32)] * 2,
            [pltpu.VMEM((_GRP,), jnp.float32)] * 2,
            [pltpu.VMEM((_SUP, _CW), jnp.float32)] * 2,
            pltpu.VMEM((_ZB, _CW), jnp.float32),
            pltpu.VMEM_SHARED((_NP, _CW), jnp.float32),
            [pltpu.SemaphoreType.DMA] * 2,
            [pltpu.SemaphoreType.DMA] * 2,
            pltpu.SemaphoreType.DMA,
        ],
    )
    def k(table_h, col_h, row_h, val_h, out_h, colv, rowv, valv, rows, zb,
          acc, semg, semsc, semi):
        core = lax.axis_index("c")
        sub = lax.axis_index("s")
        zvec = jnp.zeros((16,), jnp.float32)

        def zfill(r, _):
            zb[r, pl.ds(0, 16)] = zvec
            zb[r, pl.ds(16, 16)] = zvec
            return 0
        lax.fori_loop(0, _ZB, zfill, 0)

        def add_cc(colv_b, ccvec):
            for k16 in range(_GRP // 16):
                colv_b[pl.ds(k16 * 16, 16)] = (
                    colv_b[pl.ds(k16 * 16, 16)] + ccvec)

        def fire_gather(p, q, off):
            for a in range(2):
                pltpu.async_copy(
                    table_h.at[colv[q].at[pl.ds(off + a * 128, 128)]],
                    rows[p].at[pl.ds(a * 128, 128)], semg[p])

        def consume(p, q, w):
            pltpu.make_async_copy(table_h.at[pl.ds(0, _SUP)], rows[p],
                                  semg[p]).wait()

            def scale(k16, _):
                v16 = valv[q][pl.ds(w * _SUP + k16 * 16, 16)]
                for l in range(16):
                    e = k16 * 16 + l
                    b = jnp.full((16,), v16[l], jnp.float32)
                    rows[p][e, pl.ds(0, 16)] = rows[p][e, pl.ds(0, 16)] * b
                    rows[p][e, pl.ds(16, 16)] = rows[p][e, pl.ds(16, 16)] * b
                return 0
            lax.fori_loop(0, _SUP // 16, scale, 0)

            for a in range(2):
                pltpu.async_copy(rows[p].at[pl.ds(a * 128, 128)],
                                 acc.at[rowv[q].at[w * 2 + a]], semsc[p],
                                 add=True)

        for j in range(_CC // _NC):
            cc = core * (_CC // _NC) + j
            for z in range(_NZ):
                pltpu.sync_copy(zb, acc.at[pl.ds(sub * _ROWS_T + z * _ZB, _ZB)])
            plsc.subcore_barrier()

            ccvec = jnp.full((16,), cc * _NP, jnp.int32)
            ebase0 = sub * _TILE_E
            rbase0 = sub * _TILE_R8

            pltpu.sync_copy(col_h.at[pl.ds(ebase0, _GRP)], colv[0])
            pltpu.sync_copy(val_h.at[pl.ds(ebase0, _GRP)], valv[0])
            pltpu.sync_copy(row_h.at[pl.ds(rbase0, 8)], rowv[0])
            add_cc(colv[0], ccvec)
            fire_gather(0, 0, 0)

            def body(sb, _):
                w = sb % 4
                g = sb // 4

                for q in range(2):
                    cond = ((w == 3) & (sb < (_NSUP - 1))
                            & (((g + 1) % 2) == q))

                    def _drain_idx(q=q):
                        pltpu.make_async_copy(col_h.at[pl.ds(0, _GRP)],
                                              colv[q], semi).wait()
                        pltpu.make_async_copy(val_h.at[pl.ds(0, _GRP)],
                                              valv[q], semi).wait()
                        pltpu.make_async_copy(row_h.at[pl.ds(0, 8)],
                                              rowv[q], semi).wait()
                        add_cc(colv[q], ccvec)
                    pl.when(cond)(_drain_idx)

                for p in range(2):
                    cond = ((sb < (_NSUP - 1)) & (sb >= 1)
                            & (((sb + 1) % 2) == p))

                    def _drain_sc(p=p):
                        pltpu.make_async_copy(table_h.at[pl.ds(0, _SUP)],
                                              rows[p], semsc[p]).wait()
                    pl.when(cond)(_drain_sc)

                for p in range(2):
                    for q in range(2):
                        cond = ((sb < (_NSUP - 1))
                                & (((sb + 1) % 2) == p)
                                & ((((sb + 1) // 4) % 2) == q))
                        off = ((sb + 1) % 4) * _SUP
                        pl.when(cond)(functools.partial(fire_gather, p, q, off))

                for q in range(2):
                    cond = (w == 0) & (sb < (_NSUP - 4)) & (((g + 1) % 2) == q)

                    def _fire_idx(q=q):
                        eb = ebase0 + (g + 1) * _GRP
                        pltpu.async_copy(col_h.at[pl.ds(eb, _GRP)],
                                         colv[q], semi)
                        pltpu.async_copy(val_h.at[pl.ds(eb, _GRP)],
                                         valv[q], semi)
                        rb = rbase0 + (g + 1) * 8
                        pltpu.async_copy(row_h.at[pl.ds(rb, 8)], rowv[q], semi)
                    pl.when(cond)(_fire_idx)

                for p in range(2):
                    for q in range(2):
                        pl.when(((sb % 2) == p) & ((g % 2) == q))(
                            functools.partial(consume, p, q, w))
                return 0
            lax.fori_loop(0, _NSUP, body, 0)
            for p in range(2):
                pltpu.make_async_copy(table_h.at[pl.ds(0, _SUP)],
                                      rows[p], semsc[p]).wait()
            plsc.subcore_barrier()

            obase = cc * _NP + sub * _ROWS_T
            for z in range(_NZ):
                pltpu.sync_copy(acc.at[pl.ds(sub * _ROWS_T + z * _ZB, _ZB)],
                                out_h.at[pl.ds(obase + z * _ZB, _ZB)])

    return k(table, col1d, row2d, val1d)


def _cat4(xs_ref):
    return jnp.concatenate([xs_ref[c] for c in range(_CC)], axis=1)


def _tc_matmul_chunks(x, w):
    def body(x_ref, w_ref, out_ref):
        xw = jnp.dot(x_ref[...], w_ref[...],
                     preferred_element_type=jnp.float32)
        for c in range(_CC):
            out_ref[c] = xw[:, c * _CW:(c + 1) * _CW]
    return _PCALL(
        body,
        grid=(_NBLK,),
        in_specs=[pl.BlockSpec((_ROWB, _EMB), lambda r: (r, 0)),
                  pl.BlockSpec((_EMB, _P), lambda r: (0, 0))],
        out_specs=pl.BlockSpec((_CC, _ROWB, _CW), lambda r: (0, r, 0)),
        out_shape=jax.ShapeDtypeStruct((_CC, _NP, _CW), jnp.float32),
    )(x, w)


def _tc_c1(xs4, adjT, wi1, wi2):
    def body(xs_ref, adjt_ref, wi1_ref, wi2_ref, H1_ref, h1_ref):
        xsb = _cat4(xs_ref)
        t = jnp.dot(xsb, wi1_ref[...], preferred_element_type=jnp.float32)
        t = jnp.maximum(t + xsb, 0.0)
        logits = jnp.dot(t, wi2_ref[...], preferred_element_type=jnp.float32)
        cols = lax.broadcasted_iota(jnp.int32, logits.shape, 1)
        logits = jnp.where(cols < _EMB, logits, -1e30)
        m = jnp.max(logits, axis=1, keepdims=True)
        p = jnp.exp(logits - m)
        H1 = p / jnp.sum(p, axis=1, keepdims=True)
        H1_ref[...] = H1
        adjtb = jnp.pad(adjt_ref[...], ((0, 0), (0, _P - _EMB)))
        g = H1 * adjtb
        s = jnp.sum(g, axis=1, keepdims=True)
        gn = g / (s + 1e-8)
        part = lax.dot_general(gn, xsb, (((0,), (0,)), ((), ())),
                               preferred_element_type=jnp.float32)

        @pl.when(pl.program_id(0) == 0)
        def _init():
            h1_ref[...] = jnp.zeros_like(h1_ref)
        h1_ref[...] += part

    return _PCALL(
        body,
        grid=(_NBLK,),
        in_specs=[pl.BlockSpec((_CC, _ROWB, _CW), lambda r: (0, r, 0)),
                  pl.BlockSpec((_ROWB, _EMB), lambda r: (r, 0)),
                  pl.BlockSpec((_P, _P), lambda r: (0, 0)),
                  pl.BlockSpec((_P, _P), lambda r: (0, 0))],
        out_specs=[pl.BlockSpec((_ROWB, _P), lambda r: (r, 0)),
                   pl.BlockSpec((_P, _P), lambda r: (0, 0))],
        out_shape=[jax.ShapeDtypeStruct((_N, _P), jnp.float32),
                   jax.ShapeDtypeStruct((_P, _P), jnp.float32)],
    )(xs4, adjT, wi1, wi2)


def _nrm(v):
    n = jnp.sqrt(jnp.sum(v * v, axis=1, keepdims=True))
    return v / jnp.maximum(n, 1e-12)


def _tc_c2_fused(H1, h1, xs4, emb, wnext):
    def body(H1_ref, h1_ref, xs_ref, emb_ref, wn_ref,
             xw_ref, accio_ref, accho_ref):
        xsb = _cat4(xs_ref)
        h2 = jnp.dot(H1_ref[...], h1_ref[...],
                     preferred_element_type=jnp.float32)
        xo = h2 + xsb
        embb = jnp.pad(emb_ref[...], ((0, 0), (0, _P - _EMB)))
        accio_ref[...] = embb + _nrm(xo)
        accho_ref[...] = _nrm(h2)
        xw = jnp.dot(xo, wn_ref[...], preferred_element_type=jnp.float32)
        for c in range(_CC):
            xw_ref[c] = xw[:, c * _CW:(c + 1) * _CW]

    return _PCALL(
        body,
        grid=(_NBLK,),
        in_specs=[pl.BlockSpec((_ROWB, _P), lambda r: (r, 0)),
                  pl.BlockSpec((_P, _P), lambda r: (0, 0)),
                  pl.BlockSpec((_CC, _ROWB, _CW), lambda r: (0, r, 0)),
                  pl.BlockSpec((_ROWB, _EMB), lambda r: (r, 0)),
                  pl.BlockSpec((_P, _P), lambda r: (0, 0))],
        out_specs=[pl.BlockSpec((_CC, _ROWB, _CW), lambda r: (0, r, 0)),
                   pl.BlockSpec((_ROWB, _P), lambda r: (r, 0)),
                   pl.BlockSpec((_ROWB, _P), lambda r: (r, 0))],
        out_shape=[jax.ShapeDtypeStruct((_CC, _NP, _CW), jnp.float32),
                   jax.ShapeDtypeStruct((_N, _P), jnp.float32),
                   jax.ShapeDtypeStruct((_N, _P), jnp.float32)],
    )(H1, h1, xs4, emb, wnext)


def _tc_c2_final(H1, h1, xs4, acci, acch):
    def body(H1_ref, h1_ref, xs_ref, acci_ref, acch_ref,
             item_ref, hs_ref):
        xsb = _cat4(xs_ref)
        h2 = jnp.dot(H1_ref[...], h1_ref[...],
                     preferred_element_type=jnp.float32)
        xo = h2 + xsb
        item = (acci_ref[...] + _nrm(xo)) * (1.0 / 3.0)
        hs = (acch_ref[...] + _nrm(h2)) * 0.5
        item_ref[...] = item[:, :_EMB]
        hs_ref[...] = hs[:, :_EMB]

    return _PCALL(
        body,
        grid=(_NBLK,),
        in_specs=[pl.BlockSpec((_ROWB, _P), lambda r: (r, 0)),
                  pl.BlockSpec((_P, _P), lambda r: (0, 0)),
                  pl.BlockSpec((_CC, _ROWB, _CW), lambda r: (0, r, 0)),
                  pl.BlockSpec((_ROWB, _P), lambda r: (r, 0)),
                  pl.BlockSpec((_ROWB, _P), lambda r: (r, 0))],
        out_specs=[pl.BlockSpec((_ROWB, _EMB), lambda r: (r, 0)),
                   pl.BlockSpec((_ROWB, _EMB), lambda r: (r, 0))],
        out_shape=[jax.ShapeDtypeStruct((_N, _EMB), jnp.float32),
                   jax.ShapeDtypeStruct((_N, _EMB), jnp.float32)],
    )(H1, h1, xs4, acci, acch)


def _pad2(w):
    return jnp.pad(w.astype(jnp.float32),
                   ((0, _P - w.shape[0]), (0, _P - w.shape[1])))


def _pad_cols(w):
    return jnp.pad(w.astype(jnp.float32), ((0, 0), (0, _P - w.shape[1])))


def kernel(adj, edge_index, edge_val, embedding, channel,
           W_item0, W_item1, W_i1, W_i2):
    del channel
    f32 = jnp.float32
    emb = embedding.astype(f32)
    W0p = _pad_cols(W_item0)
    W1p = _pad2(W_item1)
    Wi1p, Wi2p = _pad2(W_i1), _pad2(W_i2)
    adjT = adj.T.astype(f32)

    rowp = jnp.pad(edge_index[0], (0, _EPAD - _E)).reshape(_R128, 128)
    colp = jnp.pad(edge_index[1], (0, _EPAD - _E))
    valp = jnp.pad(edge_val.astype(f32), (0, _EPAD - _E))

    xw = _tc_matmul_chunks(emb, W0p)
    acci = acch = item = hs = None
    for i in range(2):
        xs4 = _sc_edge_segsum(xw.reshape(_CC * _NP, _CW), colp, rowp, valp)
        xs4 = xs4.reshape(_CC, _NP, _CW)
        H1, h1 = _tc_c1(xs4, adjT, Wi1p, Wi2p)
        if i == 0:
            xw, acci, acch = _tc_c2_fused(H1, h1, xs4, emb, W1p)
        else:
            item, hs = _tc_c2_final(H1, h1, xs4, acci, acch)
    return item, hs


# speedup vs baseline: 6.7095x; 1.1899x over previous
import functools

import jax
import jax.numpy as jnp
from jax import lax
from jax.experimental import pallas as pl
from jax.experimental.pallas import tpu as pltpu
from jax.experimental.pallas import tpu_sc as plsc

_PCALL = pl.pallas_call

_N = 50000
_E = 800000
_EMB = 100
_P = 128
_CC = 4
_CW = 32
_NC = 2
_NS = 16
_NP = 51200
_EPAD = 802816
_R128 = _EPAD // 128
_TILE_E = _EPAD // _NS
_SUP = 256
_GRP = 1024
_NSUP = _TILE_E // _SUP
_NGRP = _TILE_E // _GRP
_TILE_R8 = _TILE_E // 128
_ROWS_T = _NP // _NS
_ZB = 160
_NZ = _ROWS_T // _ZB
_ROWB = 2000
_NBLK = _N // _ROWB


def _sc_edge_segsum(table, col1d, row2d, val1d):
    mesh = plsc.VectorSubcoreMesh(
        core_axis_name="c", subcore_axis_name="s",
        num_cores=_NC, num_subcores=_NS)

    @functools.partial(
        pl.kernel,
        out_type=jax.ShapeDtypeStruct((_NP, _P), jnp.float32),
        mesh=mesh,
        compiler_params=pltpu.CompilerParams(use_tc_tiling_on_sc=False),
        scratch_types=[
            [pltpu.VMEM((_GRP,), jnp.int32)] * 2,
            [pltpu.VMEM((8, 128), jnp.int32)] * 2,
            [pltpu.VMEM((_GRP,), jnp.float32)] * 2,
            [pltpu.VMEM((_SUP, _CW), jnp.float32)] * 2,
            pltpu.VMEM((_ZB, _CW), jnp.float32),
            pltpu.VMEM_SHARED((_NP, _CW), jnp.float32),
            [pltpu.SemaphoreType.DMA] * 2,
            [pltpu.SemaphoreType.DMA] * 2,
            pltpu.SemaphoreType.DMA,
        ],
    )
    def k(table_h, col_h, row_h, val_h, out_h, colv, rowv, valv, rows, zb,
          acc, semg, semsc, semi):
        core = lax.axis_index("c")
        sub = lax.axis_index("s")
        zvec = jnp.zeros((16,), jnp.float32)

        def zfill(r, _):
            zb[r, pl.ds(0, 16)] = zvec
            zb[r, pl.ds(16, 16)] = zvec
            return 0
        lax.fori_loop(0, _ZB, zfill, 0)

        def add_cc(colv_b, ccvec):
            for k16 in range(_GRP // 16):
                colv_b[pl.ds(k16 * 16, 16)] = (
                    colv_b[pl.ds(k16 * 16, 16)] * 4 + ccvec)

        def fire_gather(p, q, off):
            for a in range(2):
                pltpu.async_copy(
                    table_h.at[colv[q].at[pl.ds(off + a * 128, 128)]],
                    rows[p].at[pl.ds(a * 128, 128)], semg[p])

        def consume(p, q, w):
            pltpu.make_async_copy(table_h.at[pl.ds(0, _SUP)], rows[p],
                                  semg[p]).wait()

            def scale(k16, _):
                v16 = valv[q][pl.ds(w * _SUP + k16 * 16, 16)]
                for l in range(16):
                    e = k16 * 16 + l
                    b = jnp.full((16,), v16[l], jnp.float32)
                    rows[p][e, pl.ds(0, 16)] = rows[p][e, pl.ds(0, 16)] * b
                    rows[p][e, pl.ds(16, 16)] = rows[p][e, pl.ds(16, 16)] * b
                return 0
            lax.fori_loop(0, _SUP // 16, scale, 0)

            for a in range(2):
                pltpu.async_copy(rows[p].at[pl.ds(a * 128, 128)],
                                 acc.at[rowv[q].at[w * 2 + a]], semsc[p],
                                 add=True)

        for j in range(_CC // _NC):
            cc = core * (_CC // _NC) + j
            for z in range(_NZ):
                pltpu.sync_copy(zb, acc.at[pl.ds(sub * _ROWS_T + z * _ZB, _ZB)])
            plsc.subcore_barrier()

            ccvec = jnp.full((16,), cc, jnp.int32)
            ebase0 = sub * _TILE_E
            rbase0 = sub * _TILE_R8

            pltpu.sync_copy(col_h.at[pl.ds(ebase0, _GRP)], colv[0])
            pltpu.sync_copy(val_h.at[pl.ds(ebase0, _GRP)], valv[0])
            pltpu.sync_copy(row_h.at[pl.ds(rbase0, 8)], rowv[0])
            add_cc(colv[0], ccvec)
            fire_gather(0, 0, 0)

            def body(sb, _):
                w = sb % 4
                g = sb // 4

                for q in range(2):
                    cond = ((w == 3) & (sb < (_NSUP - 1))
                            & (((g + 1) % 2) == q))

                    def _drain_idx(q=q):
                        pltpu.make_async_copy(col_h.at[pl.ds(0, _GRP)],
                                              colv[q], semi).wait()
                        pltpu.make_async_copy(val_h.at[pl.ds(0, _GRP)],
                                              valv[q], semi).wait()
                        pltpu.make_async_copy(row_h.at[pl.ds(0, 8)],
                                              rowv[q], semi).wait()
                        add_cc(colv[q], ccvec)
                    pl.when(cond)(_drain_idx)

                for p in range(2):
                    cond = ((sb < (_NSUP - 1)) & (sb >= 1)
                            & (((sb + 1) % 2) == p))

                    def _drain_sc(p=p):
                        pltpu.make_async_copy(table_h.at[pl.ds(0, _SUP)],
                                              rows[p], semsc[p]).wait()
                    pl.when(cond)(_drain_sc)

                for p in range(2):
                    for q in range(2):
                        cond = ((sb < (_NSUP - 1))
                                & (((sb + 1) % 2) == p)
                                & ((((sb + 1) // 4) % 2) == q))
                        off = ((sb + 1) % 4) * _SUP
                        pl.when(cond)(functools.partial(fire_gather, p, q, off))

                for q in range(2):
                    cond = (w == 0) & (sb < (_NSUP - 4)) & (((g + 1) % 2) == q)

                    def _fire_idx(q=q):
                        eb = ebase0 + (g + 1) * _GRP
                        pltpu.async_copy(col_h.at[pl.ds(eb, _GRP)],
                                         colv[q], semi)
                        pltpu.async_copy(val_h.at[pl.ds(eb, _GRP)],
                                         valv[q], semi)
                        rb = rbase0 + (g + 1) * 8
                        pltpu.async_copy(row_h.at[pl.ds(rb, 8)], rowv[q], semi)
                    pl.when(cond)(_fire_idx)

                for p in range(2):
                    for q in range(2):
                        pl.when(((sb % 2) == p) & ((g % 2) == q))(
                            functools.partial(consume, p, q, w))
                return 0
            lax.fori_loop(0, _NSUP, body, 0)
            for p in range(2):
                pltpu.make_async_copy(table_h.at[pl.ds(0, _SUP)],
                                      rows[p], semsc[p]).wait()
            plsc.subcore_barrier()

            for z in range(_NZ):
                rb = sub * _ROWS_T + z * _ZB
                pltpu.sync_copy(acc.at[pl.ds(rb, _ZB)],
                                out_h.at[pl.ds(rb, _ZB), pl.ds(cc * _CW, _CW)])

    return k(table, col1d, row2d, val1d)


def _tc_matmul_chunks(x, w):
    def body(x_ref, w_ref, out_ref):
        out_ref[...] = jnp.dot(x_ref[...], w_ref[...],
                               preferred_element_type=jnp.float32)
    return _PCALL(
        body,
        grid=(_NBLK,),
        in_specs=[pl.BlockSpec((_ROWB, _EMB), lambda r: (r, 0)),
                  pl.BlockSpec((_EMB, _P), lambda r: (0, 0))],
        out_specs=pl.BlockSpec((_ROWB, _P), lambda r: (r, 0)),
        out_shape=jax.ShapeDtypeStruct((_NP, _P), jnp.float32),
    )(x, w)


def _tc_c1(xs4, adjT, wi1, wi2, rowb=1000):
    def body(xs_ref, adjt_ref, wi1_ref, wi2_ref, H1_ref, h1_ref):
        xsb = xs_ref[...]
        t = jnp.dot(xsb, wi1_ref[...], preferred_element_type=jnp.float32)
        t = jnp.maximum(t + xsb, 0.0)
        logits = jnp.dot(t, wi2_ref[...], preferred_element_type=jnp.float32)
        cols = lax.broadcasted_iota(jnp.int32, logits.shape, 1)
        logits = jnp.where(cols < _EMB, logits, -1e30)
        m = jnp.max(logits, axis=1, keepdims=True)
        p = jnp.exp(logits - m)
        H1 = p / jnp.sum(p, axis=1, keepdims=True)
        H1_ref[...] = H1
        g = H1 * adjt_ref[...]
        s = jnp.sum(g, axis=1, keepdims=True)
        gn = g / (s + 1e-8)
        part = lax.dot_general(gn, xsb, (((0,), (0,)), ((), ())),
                               preferred_element_type=jnp.float32)

        @pl.when(pl.program_id(0) == 0)
        def _init():
            h1_ref[...] = jnp.zeros_like(h1_ref)
        h1_ref[...] += part

    return _PCALL(
        body,
        grid=(_N // rowb,),
        in_specs=[pl.BlockSpec((rowb, _P), lambda r: (r, 0)),
                  pl.BlockSpec((rowb, _P), lambda r: (r, 0)),
                  pl.BlockSpec((_P, _P), lambda r: (0, 0)),
                  pl.BlockSpec((_P, _P), lambda r: (0, 0))],
        out_specs=[pl.BlockSpec((rowb, _P), lambda r: (r, 0)),
                   pl.BlockSpec((_P, _P), lambda r: (0, 0))],
        out_shape=[jax.ShapeDtypeStruct((_N, _P), jnp.float32),
                   jax.ShapeDtypeStruct((_P, _P), jnp.float32)],
    )(xs4, adjT, wi1, wi2)


def _nrm(v):
    n = jnp.sqrt(jnp.sum(v * v, axis=1, keepdims=True))
    return v / jnp.maximum(n, 1e-12)


def _tc_c2_fused(H1, h1, xs4, emb, wnext):
    def body(H1_ref, h1_ref, xs_ref, emb_ref, wn_ref,
             xw_ref, accio_ref, accho_ref):
        xsb = xs_ref[...]
        h2 = jnp.dot(H1_ref[...], h1_ref[...],
                     preferred_element_type=jnp.float32)
        xo = h2 + xsb
        embb = jnp.pad(emb_ref[...], ((0, 0), (0, _P - _EMB)))
        accio_ref[...] = embb + _nrm(xo)
        accho_ref[...] = _nrm(h2)
        xw_ref[...] = jnp.dot(xo, wn_ref[...],
                              preferred_element_type=jnp.float32)

    return _PCALL(
        body,
        grid=(_NBLK,),
        in_specs=[pl.BlockSpec((_ROWB, _P), lambda r: (r, 0)),
                  pl.BlockSpec((_P, _P), lambda r: (0, 0)),
                  pl.BlockSpec((_ROWB, _P), lambda r: (r, 0)),
                  pl.BlockSpec((_ROWB, _EMB), lambda r: (r, 0)),
                  pl.BlockSpec((_P, _P), lambda r: (0, 0))],
        out_specs=[pl.BlockSpec((_ROWB, _P), lambda r: (r, 0)),
                   pl.BlockSpec((_ROWB, _P), lambda r: (r, 0)),
                   pl.BlockSpec((_ROWB, _P), lambda r: (r, 0))],
        out_shape=[jax.ShapeDtypeStruct((_NP, _P), jnp.float32),
                   jax.ShapeDtypeStruct((_N, _P), jnp.float32),
                   jax.ShapeDtypeStruct((_N, _P), jnp.float32)],
    )(H1, h1, xs4, emb, wnext)


def _tc_c2_final(H1, h1, xs4, acci, acch):
    def body(H1_ref, h1_ref, xs_ref, acci_ref, acch_ref,
             item_ref, hs_ref):
        xsb = xs_ref[...]
        h2 = jnp.dot(H1_ref[...], h1_ref[...],
                     preferred_element_type=jnp.float32)
        xo = h2 + xsb
        item = (acci_ref[...] + _nrm(xo)) * (1.0 / 3.0)
        hs = (acch_ref[...] + _nrm(h2)) * 0.5
        item_ref[...] = item[:, :_EMB]
        hs_ref[...] = hs[:, :_EMB]

    return _PCALL(
        body,
        grid=(_NBLK,),
        in_specs=[pl.BlockSpec((_ROWB, _P), lambda r: (r, 0)),
                  pl.BlockSpec((_P, _P), lambda r: (0, 0)),
                  pl.BlockSpec((_ROWB, _P), lambda r: (r, 0)),
                  pl.BlockSpec((_ROWB, _P), lambda r: (r, 0)),
                  pl.BlockSpec((_ROWB, _P), lambda r: (r, 0))],
        out_specs=[pl.BlockSpec((_ROWB, _EMB), lambda r: (r, 0)),
                   pl.BlockSpec((_ROWB, _EMB), lambda r: (r, 0))],
        out_shape=[jax.ShapeDtypeStruct((_N, _EMB), jnp.float32),
                   jax.ShapeDtypeStruct((_N, _EMB), jnp.float32)],
    )(H1, h1, xs4, acci, acch)


def _pad2(w):
    return jnp.pad(w.astype(jnp.float32),
                   ((0, _P - w.shape[0]), (0, _P - w.shape[1])))


def _pad_cols(w):
    return jnp.pad(w.astype(jnp.float32), ((0, 0), (0, _P - w.shape[1])))


def kernel(adj, edge_index, edge_val, embedding, channel,
           W_item0, W_item1, W_i1, W_i2):
    del channel
    f32 = jnp.float32
    emb = embedding.astype(f32)
    W0p = _pad_cols(W_item0)
    W1p = _pad2(W_item1)
    Wi1p, Wi2p = _pad2(W_i1), _pad2(W_i2)
    adjT = jnp.pad(adj.T.astype(f32), ((0, 0), (0, _P - _EMB)))

    rowp = jnp.pad(edge_index[0], (0, _EPAD - _E)).reshape(_R128, 128)
    colp = jnp.pad(edge_index[1], (0, _EPAD - _E))
    valp = jnp.pad(edge_val.astype(f32), (0, _EPAD - _E))

    xw = _tc_matmul_chunks(emb, W0p)
    acci = acch = item = hs = None
    for i in range(2):
        xs = _sc_edge_segsum(xw.reshape(_CC * _NP, _CW), colp, rowp, valp)
        H1, h1 = _tc_c1(xs, adjT, Wi1p, Wi2p)
        if i == 0:
            xw, acci, acch = _tc_c2_fused(H1, h1, xs, emb, W1p)
        else:
            item, hs = _tc_c2_final(H1, h1, xs, acci, acch)
    return item, hs
